# Initial kernel scaffold; baseline (speedup 1.0000x reference)
import jax
import jax.numpy as jnp
from jax.experimental import pallas as pl


def kernel(x, boundary_points):
    raise NotImplementedError("write your pallas kernel here")


# speedup vs baseline: 2.2898x; 2.2898x over previous
import functools

import jax
import jax.numpy as jnp
from jax.experimental import pallas as pl

_BQ = 1024
_BK = 4096


def _min_dist_kernel(x_ref, bt_ref, o_ref):
    qx = x_ref[:, 0:1]
    qy = x_ref[:, 1:2]
    bx = bt_ref[0:1, :]
    by = bt_ref[1:2, :]
    dx = qx - bx
    dy = qy - by
    d2 = dx * dx + dy * dy
    o_ref[:, :] = jnp.sqrt(jnp.min(d2, axis=1, keepdims=True))


@jax.jit
def kernel(x, boundary_points):
    q = x.shape[0]
    bt = boundary_points.T
    min_dist = pl.pallas_call(
        _min_dist_kernel,
        grid=(q // _BQ,),
        in_specs=[
            pl.BlockSpec((_BQ, 2), lambda i: (i, 0)),
            pl.BlockSpec(bt.shape, lambda i: (0, 0)),
        ],
        out_specs=pl.BlockSpec((_BQ, 1), lambda i: (i, 0)),
        out_shape=jax.ShapeDtypeStruct((q, 1), x.dtype),
    )(x, bt)
    return jnp.concatenate([x, min_dist], axis=-1)
